# baseline (speedup 1.0000x reference)
import functools

import jax
import jax.numpy as jnp
from jax import lax
from jax.experimental import pallas as pl
from jax.experimental.pallas import tpu as pltpu
from jax.experimental.pallas import tpu_sc as plsc

_N = 10000
_NE = 320000
_D = 128
_DE = 144
_DC = 2 * _DE
_NC = 2
_NS = 16
_L = 16
_NW = _NC * _NS
_EPT = _NE // _NW
_CHUNK = 64
_BLK = 1000
_GRID = _N // _BLK

_f32 = jnp.float32


def _proj_body(xw_ref, xs_ref, w1_ref, b1_ref, w2_ref, b2_ref,
               ws1_ref, wd1_ref, ws2_ref, wd2_ref,
               as1_ref, ad1_ref, as2_ref, ad2_ref,
               hw_ref, hs_ref, hcat_ref, scal_ref):
  hw = jnp.dot(xw_ref[...], w1_ref[...].T, preferred_element_type=_f32)
  hw = hw + b1_ref[...]
  hs = jnp.dot(xs_ref[...], w2_ref[...].T, preferred_element_type=_f32)
  hs = hs + b2_ref[...]
  hw_ref[...] = hw
  hs_ref[...] = hs
  hs1 = jnp.dot(hw, ws1_ref[...].T, preferred_element_type=_f32)
  hd1 = jnp.dot(hs, wd1_ref[...].T, preferred_element_type=_f32)
  hs2 = jnp.dot(hs, ws2_ref[...].T, preferred_element_type=_f32)
  hd2 = jnp.dot(hw, wd2_ref[...].T, preferred_element_type=_f32)
  a_s1 = jnp.sum(hs1 * as1_ref[...], axis=1)
  a_d1 = jnp.sum(hd1 * ad1_ref[...], axis=1)
  a_s2 = jnp.sum(hs2 * as2_ref[...], axis=1)
  a_d2 = jnp.sum(hd2 * ad2_ref[...], axis=1)

  def _leaky(x):
    return jnp.where(x >= 0.0, x, 0.2 * x)

  self1 = jnp.exp(_leaky(a_s1 + a_d1))
  self2 = jnp.exp(_leaky(a_s2 + a_d2))
  ones = jnp.ones((_BLK, 1), _f32)
  zpad = jnp.zeros((_BLK, _DE - _D - 1), _f32)
  hcat_ref[...] = jnp.concatenate([hs1, ones, zpad, hs2, ones, zpad], axis=1)
  scal_ref[...] = jnp.stack(
      [a_s1, a_d1, a_s2, a_d2, self1, self2,
       jnp.zeros((_BLK,), _f32), jnp.zeros((_BLK,), _f32)], axis=1)


_row_spec = pl.BlockSpec((_BLK, _D), lambda i: (i, 0))
_cat_spec = pl.BlockSpec((_BLK, _DC), lambda i: (i, 0))
_w_spec = pl.BlockSpec((_D, _D), lambda i: (0, 0))
_v_spec = pl.BlockSpec((1, _D), lambda i: (0, 0))
_scal_spec = pl.BlockSpec((_BLK, 8), lambda i: (i, 0))

_proj_call = pl.pallas_call(
    _proj_body,
    grid=(_GRID,),
    in_specs=[_row_spec, _row_spec, _w_spec, _v_spec, _w_spec, _v_spec,
              _w_spec, _w_spec, _w_spec, _w_spec,
              _v_spec, _v_spec, _v_spec, _v_spec],
    out_specs=[_row_spec, _row_spec, _cat_spec, _scal_spec],
    out_shape=[
        jax.ShapeDtypeStruct((_N, _D), _f32),
        jax.ShapeDtypeStruct((_N, _D), _f32),
        jax.ShapeDtypeStruct((_N, _DC), _f32),
        jax.ShapeDtypeStruct((_N, 8), _f32),
    ],
)


_sc_mesh = plsc.VectorSubcoreMesh(
    core_axis_name="c", subcore_axis_name="s", num_cores=_NC, num_subcores=_NS)

_Q = 5
_QN = _N // _Q
_ACC = 2080
_DUMMY = _QN + 8
_ZPS = _ACC // _NS
_DPS = _QN // _NS

_SHIFT = 14
_MASKS = (1 << _SHIFT) - 1
_PSTG = 1024
_ARENA = _EPT + _Q * _CHUNK


def _scalar_last(v):
  return lax.squeeze(lax.slice(v, (_L - 1,), (_L,)), (0,))


@functools.partial(
    pl.kernel,
    out_type=jax.ShapeDtypeStruct((_NC, _N, _DC), _f32),
    mesh=_sc_mesh,
    scratch_types=[
        pltpu.VMEM((_PSTG,), jnp.int32),
        pltpu.VMEM((_ARENA,), jnp.int32),
        pltpu.VMEM((_N,), _f32),
        pltpu.VMEM((_N,), _f32),
        pltpu.VMEM((_N,), _f32),
        pltpu.VMEM((_N,), _f32),
        pltpu.VMEM((_CHUNK,), _f32),
        pltpu.VMEM((_CHUNK,), _f32),
        pltpu.VMEM((_CHUNK,), _f32),
        pltpu.VMEM((_CHUNK,), _f32),
        pltpu.VMEM((_CHUNK,), jnp.int32),
        pltpu.VMEM((_CHUNK,), jnp.int32),
        pltpu.VMEM((_CHUNK,), jnp.int32),
        pltpu.VMEM((_CHUNK,), jnp.int32),
        pltpu.VMEM((_CHUNK, _DC), _f32),
        pltpu.VMEM((_CHUNK, _DC), _f32),
        pltpu.VMEM_SHARED((_ACC, _DC), _f32),
        pltpu.SemaphoreType.DMA,
        pltpu.SemaphoreType.DMA,
    ],
    compiler_params=pltpu.CompilerParams(
        use_tc_tiling_on_sc=False, needs_layout_passes=False),
)
def _gat_edges(epk, a_s1, a_d1, a_s2, a_d2, hcat, zrows, out,
               pstage, arena, as1v, ad1v, as2v, ad2v,
               ex1A, ex2A, ex1B, ex2B, dadjA, dadjB, schA, schB,
               rowsA, rowsB, accum, semA, semB):
  c = lax.axis_index("c")
  s = lax.axis_index("s")
  w = c * _NS + s
  i32 = jnp.int32

  def quarter_of(dv):
    q = (dv >= _QN).astype(i32)
    for k in range(2, _Q):
      q = q + (dv >= k * _QN).astype(i32)
    return q

  nblk = -(-_EPT // _PSTG)

  def cnt_body(i, carry):
    pv = pstage[pl.ds(i * _L, _L)]
    dv = lax.shift_right_logical(pv, _SHIFT)
    q = quarter_of(dv)
    out_c = []
    for qq in range(_Q):
      pc = plsc.all_reduce_population_count(q == qq)
      out_c.append(carry[qq] + _scalar_last(pc))
    return tuple(out_c)

  cnts = (0,) * _Q
  for b in range(nblk):
    blk = min(_PSTG, _EPT - b * _PSTG)
    pltpu.sync_copy(epk.at[w, pl.ds(b * _PSTG, blk)],
                    pstage.at[pl.ds(0, blk)])
    cnts = lax.fori_loop(0, blk // _L, cnt_body, cnts)

  qstart = []
  nch_q = []
  pos = 0
  for qq in range(_Q):
    qstart.append(pos)
    nch = (cnts[qq] + _CHUNK - 1) // _CHUNK
    nch_q.append(nch)
    pos = pos + nch * _CHUNK

  def place_body(i, carry):
    pv = pstage[pl.ds(i * _L, _L)]
    dv = lax.shift_right_logical(pv, _SHIFT)
    q = quarter_of(dv)
    out_p = []
    for qq in range(_Q):
      m = q == qq
      mi = m.astype(i32)
      cs = plsc.cumsum(mi)
      plsc.store_scatter(arena, [carry[qq] + cs - mi], pv, mask=m)
      out_p.append(carry[qq] + _scalar_last(cs))
    return tuple(out_p)

  ptr = tuple(qstart)
  for b in range(nblk):
    blk = min(_PSTG, _EPT - b * _PSTG)
    pltpu.sync_copy(epk.at[w, pl.ds(b * _PSTG, blk)],
                    pstage.at[pl.ds(0, blk)])
    ptr = lax.fori_loop(0, blk // _L, place_body, ptr)

  iota = lax.iota(i32, _L)
  zv = jnp.zeros((_L,), i32)
  for qq in range(_Q):
    end = qstart[qq] + nch_q[qq] * _CHUNK
    for t in range(_CHUNK // _L):
      idx = ptr[qq] + t * _L + iota
      plsc.store_scatter(arena, [idx], zv, mask=idx < end)

  pltpu.sync_copy(a_s1, as1v)
  pltpu.sync_copy(a_d1, ad1v)
  pltpu.sync_copy(a_s2, as2v)
  pltpu.sync_copy(a_d2, ad2v)

  onehot = jnp.where(iota == 0, 1.0, 0.0)

  for q in range(_Q):
    base = q * _QN
    nch = nch_q[q]
    org = qstart[q]
    pltpu.sync_copy(zrows, accum.at[pl.ds(s * _ZPS, _ZPS)])
    plsc.subcore_barrier()

    def prep(ci, ex1, ex2, dadj, sch):
      off = pl.multiple_of(org + ci * _CHUNK, _L)
      for j in range(_CHUNK // _L):
        pv = arena[pl.ds(off + j * _L, _L)]
        sv = pv & _MASKS
        dv = lax.shift_right_logical(pv, _SHIFT)
        valid = sv != dv
        a1 = plsc.load_gather(as1v, [sv]) + plsc.load_gather(ad1v, [dv])
        a1 = jnp.where(a1 >= 0.0, a1, 0.2 * a1)
        ex1[pl.ds(j * _L, _L)] = jnp.where(valid, jnp.exp(a1), 0.0)
        a2 = plsc.load_gather(as2v, [sv]) + plsc.load_gather(ad2v, [dv])
        a2 = jnp.where(a2 >= 0.0, a2, 0.2 * a2)
        ex2[pl.ds(j * _L, _L)] = jnp.where(valid, jnp.exp(a2), 0.0)
        dvh = dv - base
        in_q = (dvh >= 0) & (dvh < _QN)
        dadj[pl.ds(j * _L, _L)] = jnp.where(in_q, dvh, _DUMMY)
        sch[pl.ds(j * _L, _L)] = sv

    def issue(sch, rows, sem):
      pltpu.async_copy(hcat.at[sch], rows, sem)

    def drain(rows, sem):
      pltpu.make_async_copy(hcat.at[pl.ds(0, _CHUNK)], rows, sem).wait()

    def consume(ex1, ex2, dadj, rows):
      @plsc.parallel_loop(0, _CHUNK, unroll=4)
      def _scale(k):
        kf = jnp.full((_L,), 0, i32) + k
        eb1 = plsc.load_gather(ex1, [kf])
        eb2 = plsc.load_gather(ex2, [kf])
        for j in range(_D // _L):
          rows[k, pl.ds(j * _L, _L)] = rows[k, pl.ds(j * _L, _L)] * eb1
        rows[k, pl.ds(_D, _L)] = eb1 * onehot
        for j in range(_D // _L):
          rows[k, pl.ds(_DE + j * _L, _L)] = (
              rows[k, pl.ds(_DE + j * _L, _L)] * eb2)
        rows[k, pl.ds(_DE + _D, _L)] = eb2 * onehot
      pltpu.sync_copy(rows, accum.at[dadj], add=True)

    @pl.when(nch > 0)
    def _prologue():
      prep(0, ex1A, ex2A, dadjA, schA)
      issue(schA, rowsA, semA)

    def pair_body(p, carry):
      i1 = 2 * p + 1
      i2 = 2 * p + 2

      @pl.when(i1 < nch)
      def _prep_b():
        prep(i1, ex1B, ex2B, dadjB, schB)
        issue(schB, rowsB, semB)

      drain(rowsA, semA)
      consume(ex1A, ex2A, dadjA, rowsA)

      @pl.when(i2 < nch)
      def _prep_a():
        prep(i2, ex1A, ex2A, dadjA, schA)
        issue(schA, rowsA, semA)

      @pl.when(i1 < nch)
      def _consume_b():
        drain(rowsB, semB)
        consume(ex1B, ex2B, dadjB, rowsB)

      return carry

    lax.fori_loop(0, (nch + 1) // 2, pair_body, 0)
    plsc.subcore_barrier()
    pltpu.sync_copy(accum.at[pl.ds(s * _DPS, _DPS)],
                    out.at[c, pl.ds(base + s * _DPS, _DPS)])
    rem = _QN - _NS * _DPS
    if rem:
      @pl.when(s == _NS - 1)
      def _dump_tail():
        pltpu.sync_copy(accum.at[pl.ds(_NS * _DPS, rem)],
                        out.at[c, pl.ds(base + _NS * _DPS, rem)])

    plsc.subcore_barrier()


def _final_body(p_ref, hcat_ref, hw_ref, hs_ref, scal_ref,
                w3_ref, b3_ref, w4_ref, b4_ref, g1b_ref, g2b_ref,
                hwo_ref, hso_ref):
  scal = scal_ref[...]
  self1 = scal[:, 4]
  self2 = scal[:, 5]
  p = p_ref[...]
  hcat = hcat_ref[...]
  hs1 = hcat[:, :_D]
  hs2 = hcat[:, _DE:_DE + _D]
  den1 = jnp.sum(p[:, :, _D:_DE], axis=(0, 2)) + self1
  num1 = jnp.sum(p[:, :, :_D], axis=0) + self1[:, None] * hs1
  nhs = num1 / den1[:, None] + g1b_ref[...]
  den2 = jnp.sum(p[:, :, _DE + _D:], axis=(0, 2)) + self2
  num2 = jnp.sum(p[:, :, _DE:_DE + _D], axis=0) + self2[:, None] * hs2
  nhw = num2 / den2[:, None] + g2b_ref[...]
  hso_ref[...] = jnp.dot(nhs + hs_ref[...], w3_ref[...].T,
                         preferred_element_type=_f32) + b3_ref[...]
  hwo_ref[...] = jnp.dot(nhw + hw_ref[...], w4_ref[...].T,
                         preferred_element_type=_f32) + b4_ref[...]


_part_spec = pl.BlockSpec((_NC, _BLK, _DC), lambda i: (0, i, 0))

_final_call = pl.pallas_call(
    _final_body,
    grid=(_GRID,),
    in_specs=[_part_spec, _cat_spec,
              _row_spec, _row_spec, _scal_spec,
              _w_spec, _v_spec, _w_spec, _v_spec, _v_spec, _v_spec],
    out_specs=[_row_spec, _row_spec],
    out_shape=[
        jax.ShapeDtypeStruct((_N, _D), _f32),
        jax.ShapeDtypeStruct((_N, _D), _f32),
    ],
)


@jax.jit
def kernel(Xw, Xs, E, W1, b1, W2, b2, g1_Wsrc, g1_Wdst, g1_as, g1_ad, g1_b,
           g2_Wsrc, g2_Wdst, g2_as, g2_ad, g2_b, W3, b3, W4, b4):
  as1 = g1_as.reshape(1, _D)
  ad1 = g1_ad.reshape(1, _D)
  as2 = g2_as.reshape(1, _D)
  ad2 = g2_ad.reshape(1, _D)
  hw, hs, hcat, scal = _proj_call(
      Xw, Xs, W1, b1.reshape(1, _D), W2, b2.reshape(1, _D),
      g1_Wsrc, g1_Wdst, g2_Wsrc, g2_Wdst, as1, ad1, as2, ad2)

  packed = (E[1] << _SHIFT) + E[0]
  epk = packed.reshape(_NW, _EPT)
  zrows = jnp.zeros((_ZPS, _DC), _f32)

  p = _gat_edges(epk, scal[:, 0], scal[:, 1], scal[:, 2], scal[:, 3],
                 hcat, zrows)

  hwo, hso = _final_call(
      p, hcat, hw, hs, scal,
      W3, b3.reshape(1, _D), W4, b4.reshape(1, _D),
      g1_b.reshape(1, _D), g2_b.reshape(1, _D))
  return hwo, hso


# speedup vs baseline: 1.1543x; 1.1543x over previous
import functools

import jax
import jax.numpy as jnp
from jax import lax
from jax.experimental import pallas as pl
from jax.experimental.pallas import tpu as pltpu
from jax.experimental.pallas import tpu_sc as plsc

_N = 10000
_NE = 320000
_D = 128
_DE = 144
_DC = 2 * _DE
_NC = 2
_NS = 16
_L = 16
_NW = _NC * _NS
_EPT = _NE // _NW
_CHUNK = 48
_BLK = 1000
_GRID = _N // _BLK

_f32 = jnp.float32


def _proj_body(xw_ref, xs_ref, w1_ref, b1_ref, w2_ref, b2_ref,
               ws1_ref, wd1_ref, ws2_ref, wd2_ref,
               as1_ref, ad1_ref, as2_ref, ad2_ref,
               hw_ref, hs_ref, hcat_ref, scal_ref):
  hw = jnp.dot(xw_ref[...], w1_ref[...].T, preferred_element_type=_f32)
  hw = hw + b1_ref[...]
  hs = jnp.dot(xs_ref[...], w2_ref[...].T, preferred_element_type=_f32)
  hs = hs + b2_ref[...]
  hw_ref[...] = hw
  hs_ref[...] = hs
  hs1 = jnp.dot(hw, ws1_ref[...].T, preferred_element_type=_f32)
  hd1 = jnp.dot(hs, wd1_ref[...].T, preferred_element_type=_f32)
  hs2 = jnp.dot(hs, ws2_ref[...].T, preferred_element_type=_f32)
  hd2 = jnp.dot(hw, wd2_ref[...].T, preferred_element_type=_f32)
  a_s1 = jnp.sum(hs1 * as1_ref[...], axis=1)
  a_d1 = jnp.sum(hd1 * ad1_ref[...], axis=1)
  a_s2 = jnp.sum(hs2 * as2_ref[...], axis=1)
  a_d2 = jnp.sum(hd2 * ad2_ref[...], axis=1)

  def _leaky(x):
    return jnp.where(x >= 0.0, x, 0.2 * x)

  self1 = jnp.exp(_leaky(a_s1 + a_d1))
  self2 = jnp.exp(_leaky(a_s2 + a_d2))
  ones = jnp.ones((_BLK, 1), _f32)
  zpad = jnp.zeros((_BLK, _DE - _D - 1), _f32)
  hcat_ref[...] = jnp.concatenate([hs1, ones, zpad, hs2, ones, zpad], axis=1)
  scal_ref[...] = jnp.stack(
      [a_s1, a_d1, a_s2, a_d2, self1, self2,
       jnp.zeros((_BLK,), _f32), jnp.zeros((_BLK,), _f32)], axis=1)


_row_spec = pl.BlockSpec((_BLK, _D), lambda i: (i, 0))
_cat_spec = pl.BlockSpec((_BLK, _DC), lambda i: (i, 0))
_w_spec = pl.BlockSpec((_D, _D), lambda i: (0, 0))
_v_spec = pl.BlockSpec((1, _D), lambda i: (0, 0))
_scal_spec = pl.BlockSpec((_BLK, 8), lambda i: (i, 0))

_proj_call = pl.pallas_call(
    _proj_body,
    grid=(_GRID,),
    in_specs=[_row_spec, _row_spec, _w_spec, _v_spec, _w_spec, _v_spec,
              _w_spec, _w_spec, _w_spec, _w_spec,
              _v_spec, _v_spec, _v_spec, _v_spec],
    out_specs=[_row_spec, _row_spec, _cat_spec, _scal_spec],
    out_shape=[
        jax.ShapeDtypeStruct((_N, _D), _f32),
        jax.ShapeDtypeStruct((_N, _D), _f32),
        jax.ShapeDtypeStruct((_N, _DC), _f32),
        jax.ShapeDtypeStruct((_N, 8), _f32),
    ],
)


_sc_mesh = plsc.VectorSubcoreMesh(
    core_axis_name="c", subcore_axis_name="s", num_cores=_NC, num_subcores=_NS)

_Q = 4
_QN = _N // _Q
_ACC = 2560
_DUMMY = _QN + 8
_ZPS = _ACC // _NS
_DPS = _QN // _NS

_SHIFT = 14
_MASKS = (1 << _SHIFT) - 1
_PSTG = 1024
_ARENA = _EPT + _Q * _CHUNK


def _scalar_last(v):
  return lax.squeeze(lax.slice(v, (_L - 1,), (_L,)), (0,))


@functools.partial(
    pl.kernel,
    out_type=jax.ShapeDtypeStruct((_NC, _N, _DC), _f32),
    mesh=_sc_mesh,
    scratch_types=[
        pltpu.VMEM((_PSTG,), jnp.int32),
        pltpu.VMEM((_ARENA,), jnp.int32),
        pltpu.VMEM((_N,), _f32),
        pltpu.VMEM((_N,), _f32),
        pltpu.VMEM((_N,), _f32),
        pltpu.VMEM((_N,), _f32),
        pltpu.VMEM((_CHUNK,), _f32),
        pltpu.VMEM((_CHUNK,), _f32),
        pltpu.VMEM((_CHUNK,), _f32),
        pltpu.VMEM((_CHUNK,), _f32),
        pltpu.VMEM((_CHUNK,), jnp.int32),
        pltpu.VMEM((_CHUNK,), jnp.int32),
        pltpu.VMEM((_CHUNK,), jnp.int32),
        pltpu.VMEM((_CHUNK,), jnp.int32),
        pltpu.VMEM((_CHUNK, _DC), _f32),
        pltpu.VMEM((_CHUNK, _DC), _f32),
        pltpu.VMEM_SHARED((_ACC, _DC), _f32),
        pltpu.SemaphoreType.DMA,
        pltpu.SemaphoreType.DMA,
    ],
    compiler_params=pltpu.CompilerParams(
        use_tc_tiling_on_sc=False, needs_layout_passes=False),
)
def _gat_edges(epk, a_s1, a_d1, a_s2, a_d2, hcat, zrows, out,
               pstage, arena, as1v, ad1v, as2v, ad2v,
               ex1A, ex2A, ex1B, ex2B, dadjA, dadjB, schA, schB,
               rowsA, rowsB, accum, semA, semB):
  c = lax.axis_index("c")
  s = lax.axis_index("s")
  w = c * _NS + s
  i32 = jnp.int32

  def quarter_of(dv):
    return ((dv >= _QN).astype(i32) + (dv >= 2 * _QN).astype(i32)
            + (dv >= 3 * _QN).astype(i32))

  nblk = -(-_EPT // _PSTG)

  def cnt_body(i, carry):
    pv = pstage[pl.ds(i * _L, _L)]
    dv = lax.shift_right_logical(pv, _SHIFT)
    q = quarter_of(dv)
    out_c = []
    for qq in range(_Q):
      pc = plsc.all_reduce_population_count(q == qq)
      out_c.append(carry[qq] + _scalar_last(pc))
    return tuple(out_c)

  cnts = (0, 0, 0, 0)
  for b in range(nblk):
    blk = min(_PSTG, _EPT - b * _PSTG)
    pltpu.sync_copy(epk.at[w, pl.ds(b * _PSTG, blk)],
                    pstage.at[pl.ds(0, blk)])
    cnts = lax.fori_loop(0, blk // _L, cnt_body, cnts)

  qstart = []
  nch_q = []
  pos = 0
  for qq in range(_Q):
    qstart.append(pos)
    nch = (cnts[qq] + _CHUNK - 1) // _CHUNK
    nch_q.append(nch)
    pos = pos + nch * _CHUNK

  def place_body(i, carry):
    pv = pstage[pl.ds(i * _L, _L)]
    dv = lax.shift_right_logical(pv, _SHIFT)
    q = quarter_of(dv)
    out_p = []
    for qq in range(_Q):
      m = q == qq
      mi = m.astype(i32)
      cs = plsc.cumsum(mi)
      plsc.store_scatter(arena, [carry[qq] + cs - mi], pv, mask=m)
      out_p.append(carry[qq] + _scalar_last(cs))
    return tuple(out_p)

  ptr = tuple(qstart)
  for b in range(nblk):
    blk = min(_PSTG, _EPT - b * _PSTG)
    pltpu.sync_copy(epk.at[w, pl.ds(b * _PSTG, blk)],
                    pstage.at[pl.ds(0, blk)])
    ptr = lax.fori_loop(0, blk // _L, place_body, ptr)

  iota = lax.iota(i32, _L)
  zv = jnp.zeros((_L,), i32)
  for qq in range(_Q):
    end = qstart[qq] + nch_q[qq] * _CHUNK
    for t in range(_CHUNK // _L):
      idx = ptr[qq] + t * _L + iota
      plsc.store_scatter(arena, [idx], zv, mask=idx < end)

  pltpu.sync_copy(a_s1, as1v)
  pltpu.sync_copy(a_d1, ad1v)
  pltpu.sync_copy(a_s2, as2v)
  pltpu.sync_copy(a_d2, ad2v)

  onehot = jnp.where(iota == 0, 1.0, 0.0)

  for q in range(_Q):
    base = q * _QN
    nch = nch_q[q]
    org = qstart[q]
    pltpu.sync_copy(zrows, accum.at[pl.ds(s * _ZPS, _ZPS)])
    plsc.subcore_barrier()

    def prep(ci, ex1, ex2, dadj, sch):
      off = pl.multiple_of(org + ci * _CHUNK, _L)
      for j in range(_CHUNK // _L):
        pv = arena[pl.ds(off + j * _L, _L)]
        sv = pv & _MASKS
        dv = lax.shift_right_logical(pv, _SHIFT)
        valid = sv != dv
        a1 = plsc.load_gather(as1v, [sv]) + plsc.load_gather(ad1v, [dv])
        a1 = jnp.where(a1 >= 0.0, a1, 0.2 * a1)
        ex1[pl.ds(j * _L, _L)] = jnp.where(valid, jnp.exp(a1), 0.0)
        a2 = plsc.load_gather(as2v, [sv]) + plsc.load_gather(ad2v, [dv])
        a2 = jnp.where(a2 >= 0.0, a2, 0.2 * a2)
        ex2[pl.ds(j * _L, _L)] = jnp.where(valid, jnp.exp(a2), 0.0)
        dvh = dv - base
        in_q = (dvh >= 0) & (dvh < _QN)
        dadj[pl.ds(j * _L, _L)] = jnp.where(in_q, dvh, _DUMMY)
        sch[pl.ds(j * _L, _L)] = sv

    def issue(sch, rows, sem):
      pltpu.async_copy(hcat.at[sch], rows, sem)

    def drain(rows, sem):
      pltpu.make_async_copy(hcat.at[pl.ds(0, _CHUNK)], rows, sem).wait()

    def consume(ex1, ex2, dadj, rows):
      @plsc.parallel_loop(0, _CHUNK, unroll=4)
      def _scale(k):
        kf = jnp.full((_L,), 0, i32) + k
        eb1 = plsc.load_gather(ex1, [kf])
        eb2 = plsc.load_gather(ex2, [kf])
        for j in range(_D // _L):
          rows[k, pl.ds(j * _L, _L)] = rows[k, pl.ds(j * _L, _L)] * eb1
        rows[k, pl.ds(_D, _L)] = eb1 * onehot
        for j in range(_D // _L):
          rows[k, pl.ds(_DE + j * _L, _L)] = (
              rows[k, pl.ds(_DE + j * _L, _L)] * eb2)
        rows[k, pl.ds(_DE + _D, _L)] = eb2 * onehot
      pltpu.sync_copy(rows, accum.at[dadj], add=True)

    @pl.when(nch > 0)
    def _prologue():
      prep(0, ex1A, ex2A, dadjA, schA)
      issue(schA, rowsA, semA)

    def pair_body(p, carry):
      i1 = 2 * p + 1
      i2 = 2 * p + 2

      @pl.when(i1 < nch)
      def _prep_b():
        prep(i1, ex1B, ex2B, dadjB, schB)
        issue(schB, rowsB, semB)

      drain(rowsA, semA)
      consume(ex1A, ex2A, dadjA, rowsA)

      @pl.when(i2 < nch)
      def _prep_a():
        prep(i2, ex1A, ex2A, dadjA, schA)
        issue(schA, rowsA, semA)

      @pl.when(i1 < nch)
      def _consume_b():
        drain(rowsB, semB)
        consume(ex1B, ex2B, dadjB, rowsB)

      return carry

    lax.fori_loop(0, (nch + 1) // 2, pair_body, 0)
    plsc.subcore_barrier()
    pltpu.sync_copy(accum.at[pl.ds(s * _DPS, _DPS)],
                    out.at[c, pl.ds(base + s * _DPS, _DPS)])
    rem = _QN - _NS * _DPS

    @pl.when(s == _NS - 1)
    def _dump_tail():
      pltpu.sync_copy(accum.at[pl.ds(_NS * _DPS, rem)],
                      out.at[c, pl.ds(base + _NS * _DPS, rem)])

    plsc.subcore_barrier()


def _final_body(p_ref, hcat_ref, hw_ref, hs_ref, scal_ref,
                w3_ref, b3_ref, w4_ref, b4_ref, g1b_ref, g2b_ref,
                hwo_ref, hso_ref):
  scal = scal_ref[...]
  self1 = scal[:, 4]
  self2 = scal[:, 5]
  p = p_ref[...]
  hcat = hcat_ref[...]
  hs1 = hcat[:, :_D]
  hs2 = hcat[:, _DE:_DE + _D]
  den1 = jnp.sum(p[:, :, _D:_DE], axis=(0, 2)) + self1
  num1 = jnp.sum(p[:, :, :_D], axis=0) + self1[:, None] * hs1
  nhs = num1 / den1[:, None] + g1b_ref[...]
  den2 = jnp.sum(p[:, :, _DE + _D:], axis=(0, 2)) + self2
  num2 = jnp.sum(p[:, :, _DE:_DE + _D], axis=0) + self2[:, None] * hs2
  nhw = num2 / den2[:, None] + g2b_ref[...]
  hso_ref[...] = jnp.dot(nhs + hs_ref[...], w3_ref[...].T,
                         preferred_element_type=_f32) + b3_ref[...]
  hwo_ref[...] = jnp.dot(nhw + hw_ref[...], w4_ref[...].T,
                         preferred_element_type=_f32) + b4_ref[...]


_part_spec = pl.BlockSpec((_NC, _BLK, _DC), lambda i: (0, i, 0))

_final_call = pl.pallas_call(
    _final_body,
    grid=(_GRID,),
    in_specs=[_part_spec, _cat_spec,
              _row_spec, _row_spec, _scal_spec,
              _w_spec, _v_spec, _w_spec, _v_spec, _v_spec, _v_spec],
    out_specs=[_row_spec, _row_spec],
    out_shape=[
        jax.ShapeDtypeStruct((_N, _D), _f32),
        jax.ShapeDtypeStruct((_N, _D), _f32),
    ],
)


@jax.jit
def kernel(Xw, Xs, E, W1, b1, W2, b2, g1_Wsrc, g1_Wdst, g1_as, g1_ad, g1_b,
           g2_Wsrc, g2_Wdst, g2_as, g2_ad, g2_b, W3, b3, W4, b4):
  as1 = g1_as.reshape(1, _D)
  ad1 = g1_ad.reshape(1, _D)
  as2 = g2_as.reshape(1, _D)
  ad2 = g2_ad.reshape(1, _D)
  hw, hs, hcat, scal = _proj_call(
      Xw, Xs, W1, b1.reshape(1, _D), W2, b2.reshape(1, _D),
      g1_Wsrc, g1_Wdst, g2_Wsrc, g2_Wdst, as1, ad1, as2, ad2)

  packed = (E[1] << _SHIFT) + E[0]
  epk = packed.reshape(_NW, _EPT)
  zrows = jnp.zeros((_ZPS, _DC), _f32)

  p = _gat_edges(epk, scal[:, 0], scal[:, 1], scal[:, 2], scal[:, 3],
                 hcat, zrows)

  hwo, hso = _final_call(
      p, hcat, hw, hs, scal,
      W3, b3.reshape(1, _D), W4, b4.reshape(1, _D),
      g1_b.reshape(1, _D), g2_b.reshape(1, _D))
  return hwo, hso
